# baseline (speedup 1.0000x reference)
import functools

import jax
import jax.numpy as jnp
from jax import lax
from jax.experimental import pallas as pl
from jax.experimental.pallas import tpu as pltpu
from jax.experimental.pallas import tpu_sc as plsc

B, W, L, D, V = 4096, 50, 20, 32, 1000
LP = L + 1
DH = D // 2
DHP = DH + 1
NW = 32
WORDS = B * W
WPT = WORDS // NW
CH = 800
NCHUNK = WPT // CH
GCH = 400
SCH = CH - GCH
SB = 4
NBATCH = SCH // SB
NBUF = 4
NMACRO = GCH // 16


def _tree_sum(vals):
    while len(vals) > 1:
        pairs = [vals[i] + vals[i + 1] for i in range(0, len(vals) - 1, 2)]
        if len(vals) % 2:
            pairs.append(vals[-1])
        vals = pairs
    return vals[0]


@functools.partial(
    pl.kernel,
    out_type=jax.ShapeDtypeStruct((WORDS * DHP,), jnp.int32),
    mesh=plsc.VectorSubcoreMesh(core_axis_name="c", subcore_axis_name="s"),
    compiler_params=pltpu.CompilerParams(needs_layout_passes=False,
                                         use_tc_tiling_on_sc=False),
    scratch_types=[
        pltpu.VMEM((V * DHP,), jnp.int32),
        pltpu.VMEM((GCH * LP,), jnp.int32),
        pltpu.VMEM((NBATCH * SB * L,), jnp.int32),
        pltpu.VMEM((CH * DHP,), jnp.int32),
    ]
    + [pltpu.VMEM((SB * L, DH), jnp.int32) for _ in range(NBUF)]
    + [pltpu.VMEM_SHARED((V, DH), jnp.int32)]
    + [pltpu.SemaphoreType.DMA for _ in range(NBUF)],
)
def _embed_sum(idsg_hbm, idss_hbm, table17_hbm, table16_hbm, out_hbm,
               table_v, idsg_v, idss_v, out_v, *bufs_and_sems):
    rows = bufs_and_sems[:NBUF]
    table_sh = bufs_and_sems[NBUF]
    sems = bufs_and_sems[NBUF + 1:]
    wid = lax.axis_index("s") * 2 + lax.axis_index("c")

    @pl.when(lax.axis_index("s") == 0)
    def _():
        pltpu.sync_copy(table16_hbm, table_sh)

    pltpu.sync_copy(table17_hbm, table_v)
    plsc.subcore_barrier()
    lane = lax.iota(jnp.int32, 16)

    def start(k, b):
        pltpu.async_copy(table_sh.at[idss_v.at[pl.ds(b * SB * L, SB * L)]],
                         rows[k], sems[k])

    def wait(k, b):
        pltpu.make_async_copy(
            table_sh.at[idss_v.at[pl.ds(b * SB * L, SB * L)]],
            rows[k], sems[k]).wait()

    def pool_stream(k, b):
        for j in range(SB):
            acc = _tree_sum([plsc.bitcast(rows[k][j * L + l], jnp.bfloat16)
                             for l in range(L)])
            out_v[pl.ds((GCH + b * SB + j) * DHP, DH)] = \
                plsc.bitcast(acc, jnp.int32)

    def gather_group(m):
        wbase = m * 16
        idpos = (wbase + lane) * LP
        rowbase = [plsc.load_gather(idsg_v, [idpos + l]) * DHP
                   for l in range(L)]
        outbase = (wbase + lane) * DHP
        for dp in range(DH):
            raw = [plsc.load_gather(table_v, [rb + dp]) for rb in rowbase]
            acc = _tree_sum([plsc.bitcast(r, jnp.bfloat16) for r in raw])
            plsc.store_scatter(out_v, [outbase + dp],
                               plsc.bitcast(acc, jnp.int32))

    def chunk_body(c, carry):
        base_word = wid * WPT + c * CH
        pltpu.sync_copy(
            idsg_hbm.at[pl.ds((wid * NCHUNK + c) * GCH * LP, GCH * LP)],
            idsg_v)
        pltpu.sync_copy(
            idss_hbm.at[pl.ds((wid * NCHUNK + c) * SCH * L, SCH * L)],
            idss_v)
        for k in range(NBUF):
            start(k, k)

        def macro(m, carry2):
            b0 = m * NBUF
            for k in range(NBUF):
                wait(k, b0 + k)
                pool_stream(k, b0 + k)
                start(k, b0 + k + NBUF)
            gather_group(m)
            return carry2

        lax.fori_loop(0, NMACRO - 1, macro, 0)
        b0 = (NMACRO - 1) * NBUF
        for k in range(NBUF):
            wait(k, b0 + k)
            pool_stream(k, b0 + k)
        gather_group(NMACRO - 1)
        pltpu.sync_copy(out_v, out_hbm.at[pl.ds(base_word * DHP, CH * DHP)])
        return carry

    lax.fori_loop(0, NCHUNK, chunk_body, 0)


def kernel(token_ids, table):
    ids = token_ids.astype(jnp.int32).reshape(NW, NCHUNK, 2, GCH, L)
    ids_g = jnp.pad(ids[:, :, 0], ((0, 0), (0, 0), (0, 0), (0, 1))
                    ).reshape(-1)
    ids_s = ids[:, :, 1].reshape(-1)
    table0 = table.at[0].set(0.0).astype(jnp.bfloat16)
    table16 = jax.lax.bitcast_convert_type(
        table0.reshape(V, DH, 2), jnp.int32)
    table17 = jnp.pad(table16, ((0, 0), (0, 1))).reshape(-1)
    out = _embed_sum(ids_g, ids_s, table17, table16)
    out = jax.lax.bitcast_convert_type(
        out.reshape(WORDS, DHP)[:, :DH], jnp.bfloat16)
    return out.astype(jnp.float32).reshape(B, W, D)


# speedup vs baseline: 1.0346x; 1.0346x over previous
import functools

import jax
import jax.numpy as jnp
from jax import lax
from jax.experimental import pallas as pl
from jax.experimental.pallas import tpu as pltpu
from jax.experimental.pallas import tpu_sc as plsc

B, W, L, D, V = 4096, 50, 20, 32, 1000
DH = D // 2
NW = 32
WORDS = B * W
WPT = WORDS // NW
CH = 800
NCHUNK = WPT // CH
SB = 5
NBATCH = CH // SB
NBUF = 4
NMACRO = NBATCH // NBUF


def _tree_sum(vals):
    while len(vals) > 1:
        pairs = [vals[i] + vals[i + 1] for i in range(0, len(vals) - 1, 2)]
        if len(vals) % 2:
            pairs.append(vals[-1])
        vals = pairs
    return vals[0]


@functools.partial(
    pl.kernel,
    out_type=jax.ShapeDtypeStruct((WORDS * DH,), jnp.int32),
    mesh=plsc.VectorSubcoreMesh(core_axis_name="c", subcore_axis_name="s"),
    compiler_params=pltpu.CompilerParams(needs_layout_passes=False,
                                         use_tc_tiling_on_sc=False),
    scratch_types=[
        pltpu.VMEM((NBATCH, SB * L), jnp.int32),
        pltpu.VMEM((CH * DH,), jnp.int32),
    ]
    + [pltpu.VMEM((SB * L, DH), jnp.int32) for _ in range(NBUF)]
    + [pltpu.VMEM_SHARED((V, DH), jnp.int32)]
    + [pltpu.SemaphoreType.DMA for _ in range(NBUF)],
)
def _embed_sum(ids_hbm, table_hbm, out_hbm, ids_v, out_v, *bufs_and_sems):
    rows = bufs_and_sems[:NBUF]
    table_sh = bufs_and_sems[NBUF]
    sems = bufs_and_sems[NBUF + 1:]
    wid = lax.axis_index("s") * 2 + lax.axis_index("c")

    @pl.when(lax.axis_index("s") == 0)
    def _():
        pltpu.sync_copy(table_hbm, table_sh)

    plsc.subcore_barrier()

    def src(k):
        return table_sh if k < 2 else table_hbm

    def start(k, b):
        pltpu.async_copy(src(k).at[ids_v.at[b]], rows[k], sems[k])

    def wait(k, b):
        pltpu.make_async_copy(src(k).at[ids_v.at[b]], rows[k],
                              sems[k]).wait()

    def pool(k, b):
        for j in range(SB):
            acc = _tree_sum([
                plsc.bitcast(rows[k][j * L + l], jnp.bfloat16)
                for l in range(L)
            ])
            out_v[pl.ds((b * SB + j) * DH, DH)] = plsc.bitcast(acc, jnp.int32)

    def chunk_body(c, carry):
        base_word = wid * WPT + c * CH
        base_batch = pl.multiple_of(base_word // SB, 8)
        pltpu.sync_copy(ids_hbm.at[pl.ds(base_batch, NBATCH)], ids_v)
        for k in range(NBUF):
            start(k, k)

        def macro(m, carry2):
            b0 = m * NBUF
            for k in range(NBUF):
                wait(k, b0 + k)
                pool(k, b0 + k)
                start(k, b0 + k + NBUF)
            return carry2

        lax.fori_loop(0, NMACRO - 1, macro, 0)
        for k in range(NBUF):
            b = (NMACRO - 1) * NBUF + k
            wait(k, b)
            pool(k, b)
        pltpu.sync_copy(out_v, out_hbm.at[pl.ds(base_word * DH, CH * DH)])
        return carry

    lax.fori_loop(0, NCHUNK, chunk_body, 0)


def kernel(token_ids, table):
    ids = token_ids.astype(jnp.int32).reshape(-1, SB * L)
    table0 = table.at[0].set(0.0).astype(jnp.bfloat16)
    table_p = jax.lax.bitcast_convert_type(
        table0.reshape(V, DH, 2), jnp.int32)
    out = _embed_sum(ids, table_p)
    out = jax.lax.bitcast_convert_type(
        out.reshape(WORDS, DH), jnp.bfloat16)
    return out.astype(jnp.float32).reshape(B, W, D)


# speedup vs baseline: 1.3160x; 1.2720x over previous
import functools

import jax
import jax.numpy as jnp
from jax import lax
from jax.experimental import pallas as pl
from jax.experimental.pallas import tpu as pltpu
from jax.experimental.pallas import tpu_sc as plsc

B, W, L, D, V = 4096, 50, 20, 32, 1000
DH = D // 2
NW = 32
WORDS = B * W
WPT = WORDS // NW
CH = 800
NCHUNK = WPT // CH
SB = 5
NBATCH = CH // SB
NBUF = 8
NMACRO = NBATCH // NBUF


def _tree_sum(vals):
    while len(vals) > 1:
        pairs = [vals[i] + vals[i + 1] for i in range(0, len(vals) - 1, 2)]
        if len(vals) % 2:
            pairs.append(vals[-1])
        vals = pairs
    return vals[0]


@functools.partial(
    pl.kernel,
    out_type=jax.ShapeDtypeStruct((WORDS * DH,), jnp.int32),
    mesh=plsc.VectorSubcoreMesh(core_axis_name="c", subcore_axis_name="s"),
    compiler_params=pltpu.CompilerParams(needs_layout_passes=False,
                                         use_tc_tiling_on_sc=False),
    scratch_types=[
        pltpu.VMEM((NBATCH, SB * L), jnp.int32),
        pltpu.VMEM((CH * DH,), jnp.int32),
    ]
    + [pltpu.VMEM((SB * L, DH), jnp.int32) for _ in range(NBUF)]
    + [pltpu.VMEM_SHARED((V, DH), jnp.int32)]
    + [pltpu.SemaphoreType.DMA for _ in range(NBUF)],
)
def _embed_sum(ids_hbm, table_hbm, out_hbm, ids_v, out_v, *bufs_and_sems):
    rows = bufs_and_sems[:NBUF]
    table_sh = bufs_and_sems[NBUF]
    sems = bufs_and_sems[NBUF + 1:]
    wid = lax.axis_index("s") * 2 + lax.axis_index("c")

    @pl.when(lax.axis_index("s") == 0)
    def _():
        pltpu.sync_copy(table_hbm, table_sh)

    plsc.subcore_barrier()

    def start(k, b):
        pltpu.async_copy(table_sh.at[ids_v.at[b]], rows[k], sems[k])

    def wait(k, b):
        pltpu.make_async_copy(table_sh.at[ids_v.at[b]], rows[k],
                              sems[k]).wait()

    def pool(k, b):
        for j in range(SB):
            acc = _tree_sum([
                plsc.bitcast(rows[k][j * L + l], jnp.bfloat16)
                for l in range(L)
            ])
            out_v[pl.ds((b * SB + j) * DH, DH)] = plsc.bitcast(acc, jnp.int32)

    def chunk_body(c, carry):
        base_word = wid * WPT + c * CH
        base_batch = pl.multiple_of(base_word // SB, 8)
        pltpu.sync_copy(ids_hbm.at[pl.ds(base_batch, NBATCH)], ids_v)
        for k in range(NBUF):
            start(k, k)

        def macro(m, carry2):
            b0 = m * NBUF
            for k in range(NBUF):
                wait(k, b0 + k)
                pool(k, b0 + k)
                start(k, b0 + k + NBUF)
            return carry2

        lax.fori_loop(0, NMACRO - 1, macro, 0)
        for k in range(NBUF):
            b = (NMACRO - 1) * NBUF + k
            wait(k, b)
            pool(k, b)
        pltpu.sync_copy(out_v, out_hbm.at[pl.ds(base_word * DH, CH * DH)])
        return carry

    lax.fori_loop(0, NCHUNK, chunk_body, 0)


def kernel(token_ids, table):
    ids = token_ids.astype(jnp.int32).reshape(-1, SB * L)
    table0 = table.at[0].set(0.0).astype(jnp.bfloat16)
    table_p = jax.lax.bitcast_convert_type(
        table0.reshape(V, DH, 2), jnp.int32)
    out = _embed_sum(ids, table_p)
    out = jax.lax.bitcast_convert_type(
        out.reshape(WORDS, DH), jnp.bfloat16)
    return out.astype(jnp.float32).reshape(B, W, D)


# speedup vs baseline: 1.3304x; 1.0109x over previous
import functools

import jax
import jax.numpy as jnp
from jax import lax
from jax.experimental import pallas as pl
from jax.experimental.pallas import tpu as pltpu
from jax.experimental.pallas import tpu_sc as plsc

B, W, L, D, V = 4096, 50, 20, 32, 1000
DH = D // 2
NW = 32
WORDS = B * W
WPT = WORDS // NW
CH = 800
NCHUNK = WPT // CH
SB = 5
NBATCH = CH // SB
NBUF = 4
NMACRO = NBATCH // NBUF


def _tree_sum(vals):
    while len(vals) > 1:
        pairs = [vals[i] + vals[i + 1] for i in range(0, len(vals) - 1, 2)]
        if len(vals) % 2:
            pairs.append(vals[-1])
        vals = pairs
    return vals[0]


@functools.partial(
    pl.kernel,
    out_type=jax.ShapeDtypeStruct((WORDS * DH,), jnp.int32),
    mesh=plsc.VectorSubcoreMesh(core_axis_name="c", subcore_axis_name="s"),
    compiler_params=pltpu.CompilerParams(needs_layout_passes=False,
                                         use_tc_tiling_on_sc=False),
    scratch_types=[
        pltpu.VMEM((NBATCH, SB * L), jnp.int32),
        pltpu.VMEM((CH * DH,), jnp.int32),
    ]
    + [pltpu.VMEM((SB * L, DH), jnp.int32) for _ in range(NBUF)]
    + [pltpu.VMEM_SHARED((V, DH), jnp.int32)]
    + [pltpu.SemaphoreType.DMA for _ in range(NBUF)],
)
def _embed_sum(ids_hbm, table_hbm, out_hbm, ids_v, out_v, *bufs_and_sems):
    rows = bufs_and_sems[:NBUF]
    table_sh = bufs_and_sems[NBUF]
    sems = bufs_and_sems[NBUF + 1:]
    wid = lax.axis_index("s") * 2 + lax.axis_index("c")

    @pl.when(lax.axis_index("s") == 0)
    def _():
        pltpu.sync_copy(table_hbm, table_sh)

    plsc.subcore_barrier()

    def start(k, b):
        pltpu.async_copy(table_sh.at[ids_v.at[b]], rows[k], sems[k])

    def wait(k, b):
        pltpu.make_async_copy(table_sh.at[ids_v.at[b]], rows[k],
                              sems[k]).wait()

    def pool(k, b):
        for j in range(SB):
            acc = _tree_sum([
                plsc.bitcast(rows[k][j * L + l], jnp.bfloat16)
                for l in range(L)
            ])
            out_v[pl.ds((b * SB + j) * DH, DH)] = plsc.bitcast(acc, jnp.int32)

    def chunk_body(c, carry):
        base_word = wid * WPT + c * CH
        base_batch = pl.multiple_of(base_word // SB, 8)
        pltpu.sync_copy(ids_hbm.at[pl.ds(base_batch, NBATCH)], ids_v)
        for k in range(NBUF):
            start(k, k)

        def macro(m, carry2):
            b0 = m * NBUF
            for k in range(NBUF):
                wait(k, b0 + k)
                pool(k, b0 + k)
                start(k, b0 + k + NBUF)
            return carry2

        lax.fori_loop(0, NMACRO - 1, macro, 0)
        for k in range(NBUF):
            b = (NMACRO - 1) * NBUF + k
            wait(k, b)
            pool(k, b)
        pltpu.sync_copy(out_v, out_hbm.at[pl.ds(base_word * DH, CH * DH)])
        return carry

    lax.fori_loop(0, NCHUNK, chunk_body, 0)


def kernel(token_ids, table):
    ids = token_ids.astype(jnp.int32).reshape(-1, SB * L)
    table0 = table.at[0].set(0.0).astype(jnp.bfloat16)
    table_p = jax.lax.bitcast_convert_type(
        table0.reshape(V, DH, 2), jnp.int32)
    out = _embed_sum(ids, table_p)
    out = jax.lax.bitcast_convert_type(
        out.reshape(WORDS, DH), jnp.bfloat16)
    return out.astype(jnp.float32).reshape(B, W, D)
